# baseline (speedup 1.0000x reference)
import functools

import jax
import jax.numpy as jnp
from jax.experimental import pallas as pl
from jax.experimental.pallas import tpu as pltpu

_VMEM_BYTES = 64 * 1024 * 1024
_BN_EPS = 1e-5


def _interleave_groups(n):
    for g in (4, 2):
        if n % g == 0:
            return g
    return 1


def _fused_body(x_ref, w_ref, gb_ref, o_ref,
                bfres_ref, mt_ref, mb_ref, st_ref, sb_ref, gate_ref,
                *, hw, h2w, n, c, r):
    p = pl.program_id(0)
    t = pl.program_id(1)
    rows = jax.lax.broadcasted_iota(jnp.int32, (n, c), 0)

    @pl.when(p == 0)
    def _phase0():
        x = x_ref[0]
        top = x[:, :h2w]
        bot = x[:, h2w:]
        sel = rows == t
        mt_ref[...] = jnp.where(sel, jnp.max(top, axis=-1)[None], mt_ref[...])
        mb_ref[...] = jnp.where(sel, jnp.max(bot, axis=-1)[None], mb_ref[...])
        st_ref[...] = jnp.where(sel, jnp.sum(top, axis=-1)[None], st_ref[...])
        sb_ref[...] = jnp.where(sel, jnp.sum(bot, axis=-1)[None], sb_ref[...])

        @pl.when(t < r)
        def _stash():
            tr = jnp.minimum(t, r - 1)
            bfres_ref[pl.ds(tr, 1)] = x_ref[...].astype(jnp.bfloat16)

    grp = _interleave_groups(n)
    gs = n // grp
    q = t // gs
    j = t % gs
    rq = (q * r) // grp
    res_q = ((q + 1) * r) // grp - rq
    from_hbm = j >= res_q
    s = jnp.where(from_hbm, r + (q * gs - rq) + (j - res_q), rq + j)

    @pl.when(p == 1)
    def _phase1():
        @pl.when(t == 0)
        def _glue():
            max_top = mt_ref[...]
            max_bot = mb_ref[...]
            sum_top = st_ref[...]
            sum_bot = sb_ref[...]
            max_all = jnp.maximum(max_top, max_bot)
            mean_all = (sum_top + sum_bot) * (1.0 / hw)
            mean_top = sum_top * (1.0 / h2w)
            mean_bot = sum_bot * (1.0 / (hw - h2w))

            w = w_ref[...]
            gamma = gb_ref[0]
            beta = gb_ref[1]

            def bn(z):
                m = jnp.mean(z, axis=0, keepdims=True)
                v = jnp.mean((z - m) ** 2, axis=0, keepdims=True)
                return gamma * (z - m) / jnp.sqrt(v + _BN_EPS) + beta

            z_max = max_all * w[0, 0] + max_top * w[0, 1] + max_top * w[0, 2]
            z_avg = mean_all * w[1, 0] + mean_top * w[1, 1] + mean_bot * w[1, 2]
            z_std = mean_all * w[2, 0] + mean_top * w[2, 1] + mean_bot * w[2, 2]
            fuse = bn(bn(z_max) * w[3, 0] + bn(z_avg) * w[3, 1] + bn(z_std) * w[3, 2])
            gate_ref[...] = jax.nn.sigmoid(fuse)

        g = jnp.sum(jnp.where(rows == s, gate_ref[...], 0.0), axis=0)
        g = g.reshape(-1, 1)

        @pl.when(jnp.logical_not(from_hbm))
        def _from_vmem():
            sr = jnp.minimum(s, r - 1)
            xb = bfres_ref[pl.ds(sr, 1)].astype(jnp.float32)
            o_ref[...] = xb * g[None]

        @pl.when(from_hbm)
        def _from_hbm():
            o_ref[...] = x_ref[...] * g[None]


def kernel(x, cfc, cfc_avg, cfc_max, cfc_std, bn_gamma, bn_beta):
    N, C, H, W = x.shape
    HW = H * W
    h2w = (H // 2) * W
    xf = x.reshape(N, C, HW)
    f32 = jnp.float32
    sample_bytes = C * HW * x.dtype.itemsize

    budget = _VMEM_BYTES - 4 * sample_bytes - 2 * 1024 * 1024
    r = max(0, min(N - 1, budget // (sample_bytes // 2)))

    w = jnp.stack([cfc_max.T, cfc_avg.T, cfc_std.T, cfc.T], 0).astype(f32)
    gb = jnp.stack([bn_gamma, bn_beta], axis=0).astype(f32)

    grp = _interleave_groups(N)
    gs = N // grp

    def _p1_maps(t):
        q = t // gs
        j = t % gs
        rq = (q * r) // grp
        res_q = ((q + 1) * r) // grp - rq
        s = jnp.where(j >= res_q, r + (q * gs - rq) + (j - res_q), rq + j)
        xi = r + (q * gs - rq) + jnp.maximum(0, j - res_q)
        return xi, s

    def x_idx(p, t):
        xi, _ = _p1_maps(t)
        return (jnp.where(p == 0, t, xi), 0, 0)

    def o_idx(p, t):
        _, s = _p1_maps(t)
        return (jnp.where(p == 0, 0, s), 0, 0)

    out = pl.pallas_call(
        functools.partial(_fused_body, hw=HW, h2w=h2w, n=N, c=C, r=r),
        out_shape=jax.ShapeDtypeStruct((N, C, HW), x.dtype),
        grid=(2, N),
        in_specs=[
            pl.BlockSpec((1, C, HW), x_idx),
            pl.BlockSpec((4, 3, C), lambda p, t: (0, 0, 0)),
            pl.BlockSpec((2, C), lambda p, t: (0, 0)),
        ],
        out_specs=pl.BlockSpec((1, C, HW), o_idx),
        scratch_shapes=[
            pltpu.VMEM((max(r, 1), C, HW), jnp.bfloat16),
            pltpu.VMEM((N, C), f32),
            pltpu.VMEM((N, C), f32),
            pltpu.VMEM((N, C), f32),
            pltpu.VMEM((N, C), f32),
            pltpu.VMEM((N, C), f32),
        ],
        compiler_params=pltpu.CompilerParams(
            dimension_semantics=("arbitrary", "arbitrary"),
            vmem_limit_bytes=_VMEM_BYTES),
    )(xf, w, gb)
    return out.reshape(N, C, H, W)


# speedup vs baseline: 1.0161x; 1.0161x over previous
import functools

import jax
import jax.numpy as jnp
from jax.experimental import pallas as pl
from jax.experimental.pallas import tpu as pltpu

_VMEM_BYTES = 64 * 1024 * 1024
_BN_EPS = 1e-5


def _fused_body(x_ref, w_ref, gb_ref, o_ref,
                bfres_ref, mt_ref, mb_ref, st_ref, sb_ref, gate_ref,
                *, hw, h2w, n, c, r):
    p = pl.program_id(0)
    t = pl.program_id(1)
    rows = jax.lax.broadcasted_iota(jnp.int32, (n, c), 0)

    @pl.when(p == 0)
    def _phase0():
        x = x_ref[0]
        top = x[:, :h2w]
        bot = x[:, h2w:]
        sel = rows == t
        mt_ref[...] = jnp.where(sel, jnp.max(top, axis=-1)[None], mt_ref[...])
        mb_ref[...] = jnp.where(sel, jnp.max(bot, axis=-1)[None], mb_ref[...])
        st_ref[...] = jnp.where(sel, jnp.sum(top, axis=-1)[None], st_ref[...])
        sb_ref[...] = jnp.where(sel, jnp.sum(bot, axis=-1)[None], sb_ref[...])

        @pl.when(t < r)
        def _stash():
            tr = jnp.minimum(t, r - 1)
            bfres_ref[pl.ds(tr, 1)] = x_ref[...].astype(jnp.bfloat16)

    s = t
    from_hbm = t >= r

    @pl.when(p == 1)
    def _phase1():
        @pl.when(t == 0)
        def _glue():
            max_top = mt_ref[...]
            max_bot = mb_ref[...]
            sum_top = st_ref[...]
            sum_bot = sb_ref[...]
            max_all = jnp.maximum(max_top, max_bot)
            mean_all = (sum_top + sum_bot) * (1.0 / hw)
            mean_top = sum_top * (1.0 / h2w)
            mean_bot = sum_bot * (1.0 / (hw - h2w))

            w = w_ref[...]
            gamma = gb_ref[0]
            beta = gb_ref[1]

            def bn(z):
                m = jnp.mean(z, axis=0, keepdims=True)
                v = jnp.mean((z - m) ** 2, axis=0, keepdims=True)
                return gamma * (z - m) / jnp.sqrt(v + _BN_EPS) + beta

            z_max = max_all * w[0, 0] + max_top * w[0, 1] + max_top * w[0, 2]
            z_avg = mean_all * w[1, 0] + mean_top * w[1, 1] + mean_bot * w[1, 2]
            z_std = mean_all * w[2, 0] + mean_top * w[2, 1] + mean_bot * w[2, 2]
            fuse = bn(bn(z_max) * w[3, 0] + bn(z_avg) * w[3, 1] + bn(z_std) * w[3, 2])
            gate_ref[...] = jax.nn.sigmoid(fuse)

        g = jnp.sum(jnp.where(rows == s, gate_ref[...], 0.0), axis=0)
        g = g.reshape(-1, 1)

        @pl.when(jnp.logical_not(from_hbm))
        def _from_vmem():
            sr = jnp.minimum(s, r - 1)
            xb = bfres_ref[pl.ds(sr, 1)].astype(jnp.float32)
            o_ref[...] = xb * g[None]

        @pl.when(from_hbm)
        def _from_hbm():
            o_ref[...] = x_ref[...] * g[None]


def kernel(x, cfc, cfc_avg, cfc_max, cfc_std, bn_gamma, bn_beta):
    N, C, H, W = x.shape
    HW = H * W
    h2w = (H // 2) * W
    xf = x.reshape(N, C, HW)
    f32 = jnp.float32
    sample_bytes = C * HW * x.dtype.itemsize

    budget = _VMEM_BYTES - 4 * sample_bytes - 2 * 1024 * 1024
    r = max(0, min(N - 1, budget // (sample_bytes // 2)))

    w = jnp.stack([cfc_max.T, cfc_avg.T, cfc_std.T, cfc.T], 0).astype(f32)
    gb = jnp.stack([bn_gamma, bn_beta], axis=0).astype(f32)

    def x_idx(p, t):
        return (jnp.where(p == 0, t, jnp.where(t < r, r, t)), 0, 0)

    def o_idx(p, t):
        return (jnp.where(p == 0, 0, t), 0, 0)

    out = pl.pallas_call(
        functools.partial(_fused_body, hw=HW, h2w=h2w, n=N, c=C, r=r),
        out_shape=jax.ShapeDtypeStruct((N, C, HW), x.dtype),
        grid=(2, N),
        in_specs=[
            pl.BlockSpec((1, C, HW), x_idx),
            pl.BlockSpec((4, 3, C), lambda p, t: (0, 0, 0)),
            pl.BlockSpec((2, C), lambda p, t: (0, 0)),
        ],
        out_specs=pl.BlockSpec((1, C, HW), o_idx),
        scratch_shapes=[
            pltpu.VMEM((max(r, 1), C, HW), jnp.bfloat16),
            pltpu.VMEM((N, C), f32),
            pltpu.VMEM((N, C), f32),
            pltpu.VMEM((N, C), f32),
            pltpu.VMEM((N, C), f32),
            pltpu.VMEM((N, C), f32),
        ],
        compiler_params=pltpu.CompilerParams(
            dimension_semantics=("arbitrary", "arbitrary"),
            vmem_limit_bytes=_VMEM_BYTES),
    )(xf, w, gb)
    return out.reshape(N, C, H, W)
